# baseline (speedup 1.0000x reference)
import functools

import jax
import jax.numpy as jnp
from jax import lax
from jax.experimental import pallas as pl
from jax.experimental.pallas import tpu as pltpu
from jax.experimental.pallas import tpu_sc as plsc

KV_HEADS = 4
HEAD_DIM = 128
GATE_DIM = 128
KV = KV_HEADS * HEAD_DIM

NUM_WORKERS = 32
GATHER_WINDOW = 32
TC_BLOCK = 2048


def _sc_gather(embed_table, ids2d):
    n = ids2d.shape[0] * ids2d.shape[1]
    per_w = n // NUM_WORKERS
    n_win = per_w // GATHER_WINDOW
    mesh = plsc.VectorSubcoreMesh(core_axis_name="c", subcore_axis_name="s")

    @functools.partial(
        pl.kernel,
        out_type=jax.ShapeDtypeStruct((n, KV), embed_table.dtype),
        mesh=mesh,
        scratch_types=[
            pltpu.VMEM((per_w,), jnp.int32),
            pltpu.VMEM((GATHER_WINDOW, KV), jnp.float32),
            pltpu.VMEM((GATHER_WINDOW, KV), jnp.float32),
            pltpu.VMEM((GATHER_WINDOW, KV), jnp.float32),
            pltpu.SemaphoreType.DMA,
            pltpu.SemaphoreType.DMA,
            pltpu.SemaphoreType.DMA,
            pltpu.SemaphoreType.DMA,
            pltpu.SemaphoreType.DMA,
            pltpu.SemaphoreType.DMA,
        ],
    )
    def gather_kernel(table_hbm, ids_hbm, out_hbm,
                      idx_v, buf0, buf1, buf2, gs0, gs1, gs2, ws0, ws1, ws2):
        wid = lax.axis_index("s") * 2 + lax.axis_index("c")
        base = wid * per_w
        w_per_row = ids_hbm.shape[1] // per_w
        pltpu.sync_copy(
            ids_hbm.at[wid // w_per_row,
                       pl.ds((wid % w_per_row) * per_w, per_w)],
            idx_v)

        bufs = (buf0, buf1, buf2)
        gsems = (gs0, gs1, gs2)
        wsems = (ws0, ws1, ws2)
        nbuf = len(bufs)
        gathers = [None] * nbuf
        writes = [None] * nbuf

        def issue_gather(w):
            b = w % nbuf
            gathers[b] = pltpu.async_copy(
                table_hbm.at[idx_v.at[pl.ds(w * GATHER_WINDOW,
                                            GATHER_WINDOW)]],
                bufs[b], gsems[b])

        for w in range(min(nbuf - 1, n_win)):
            issue_gather(w)
        for w in range(n_win):
            b = w % nbuf
            gathers[b].wait()
            wn = w + nbuf - 1
            if wn < n_win:
                bn = wn % nbuf
                if writes[bn] is not None:
                    writes[bn].wait()
                issue_gather(wn)
            writes[b] = pltpu.async_copy(
                bufs[b],
                out_hbm.at[pl.ds(base + w * GATHER_WINDOW, GATHER_WINDOW)],
                wsems[b])
        for wr in writes:
            if wr is not None:
                wr.wait()

    return gather_kernel(embed_table, ids2d)


GATE_BLOCK = 1024


def _tc_gate_kernel(x_ref, w_ref, b_ref, s_ref):
    logits = jnp.dot(x_ref[0], w_ref[...],
                     preferred_element_type=jnp.float32) + b_ref[...]
    s_ref[...] = 2.0 * jax.nn.sigmoid(logits)


def _tc_gate(x, w_t, b_row):
    bsz, seq, _ = x.shape
    t_blocks = seq // GATE_BLOCK
    return pl.pallas_call(
        _tc_gate_kernel,
        out_shape=jax.ShapeDtypeStruct((bsz * seq, KV_HEADS), jnp.float32),
        grid=(bsz, t_blocks),
        in_specs=[
            pl.BlockSpec((1, GATE_BLOCK, GATE_DIM), lambda bi, ti: (bi, ti, 0)),
            pl.BlockSpec((GATE_DIM, KV_HEADS), lambda bi, ti: (0, 0)),
            pl.BlockSpec((1, KV_HEADS), lambda bi, ti: (0, 0)),
        ],
        out_specs=pl.BlockSpec((GATE_BLOCK, KV_HEADS),
                               lambda bi, ti: (bi * t_blocks + ti, 0)),
    )(x, w_t, b_row)


def _tc_scale_kernel(ve_ref, s_ref, out_ref):
    s = s_ref[...]
    for h in range(KV_HEADS):
        out_ref[0, :, h, :] = (ve_ref[:, h * HEAD_DIM:(h + 1) * HEAD_DIM]
                               * s[:, h][:, None])


def _tc_scale_chunk_kernel(prev_ref, ve_ref, s_ref, out_ref):
    del prev_ref
    _tc_scale_kernel(ve_ref, s_ref, out_ref)


N_CHUNKS = 1


def _tc_scale_chunk(out_prev, ve_k, s, k, bsz, seq):
    ck = (bsz * seq) // N_CHUNKS
    t_blocks = ck // TC_BLOCK
    tb_per_b = seq // TC_BLOCK
    base_blk = (k * ck) // TC_BLOCK

    def out_idx(ti):
        flat = base_blk + ti
        return (flat // tb_per_b, flat % tb_per_b, 0, 0)

    ve_spec = pl.BlockSpec((TC_BLOCK, KV), lambda ti: (ti, 0))
    s_spec = pl.BlockSpec((TC_BLOCK, KV_HEADS),
                          lambda ti: (base_blk + ti, 0))
    out_spec = pl.BlockSpec((1, TC_BLOCK, KV_HEADS, HEAD_DIM), out_idx)
    out_shape = jax.ShapeDtypeStruct((bsz, seq, KV_HEADS, HEAD_DIM),
                                     jnp.float32)
    if out_prev is None:
        return pl.pallas_call(
            _tc_scale_kernel,
            out_shape=out_shape,
            grid=(t_blocks,),
            in_specs=[ve_spec, s_spec],
            out_specs=out_spec,
        )(ve_k, s)
    return pl.pallas_call(
        _tc_scale_chunk_kernel,
        out_shape=out_shape,
        grid=(t_blocks,),
        in_specs=[pl.BlockSpec(memory_space=pl.ANY), ve_spec, s_spec],
        out_specs=out_spec,
        input_output_aliases={0: 0},
    )(out_prev, ve_k, s)


def kernel(input_ids, x, layer_idx, embed_table, gate_W, gate_b):
    b, t = input_ids.shape
    n = b * t
    ck = n // N_CHUNKS

    s = _tc_gate(x, gate_W.T, gate_b.reshape(1, KV_HEADS))
    ids_flat = input_ids.reshape(1, n)
    ves = [_sc_gather(embed_table, lax.slice(ids_flat, (0, k * ck),
                                             (1, (k + 1) * ck)))
           for k in range(N_CHUNKS)]
    out = None
    for k in range(N_CHUNKS):
        out = _tc_scale_chunk(out, ves[k], s, k, b, t)
    return out


# speedup vs baseline: 1.0150x; 1.0150x over previous
import functools

import jax
import jax.numpy as jnp
from jax import lax
from jax.experimental import pallas as pl
from jax.experimental.pallas import tpu as pltpu
from jax.experimental.pallas import tpu_sc as plsc

KV_HEADS = 4
HEAD_DIM = 128
GATE_DIM = 128
KV = KV_HEADS * HEAD_DIM

NUM_WORKERS = 32
GATHER_WINDOW = 64
TC_BLOCK = 2048


def _sc_gather(embed_table, ids2d):
    n = ids2d.shape[0] * ids2d.shape[1]
    per_w = n // NUM_WORKERS
    n_win = per_w // GATHER_WINDOW
    mesh = plsc.VectorSubcoreMesh(core_axis_name="c", subcore_axis_name="s")

    @functools.partial(
        pl.kernel,
        out_type=jax.ShapeDtypeStruct((n, KV), embed_table.dtype),
        mesh=mesh,
        scratch_types=[
            pltpu.VMEM((per_w,), jnp.int32),
            pltpu.VMEM((GATHER_WINDOW, KV), jnp.float32),
            pltpu.VMEM((GATHER_WINDOW, KV), jnp.float32),
            pltpu.VMEM((GATHER_WINDOW, KV), jnp.float32),
            pltpu.SemaphoreType.DMA,
            pltpu.SemaphoreType.DMA,
            pltpu.SemaphoreType.DMA,
            pltpu.SemaphoreType.DMA,
            pltpu.SemaphoreType.DMA,
            pltpu.SemaphoreType.DMA,
        ],
    )
    def gather_kernel(table_hbm, ids_hbm, out_hbm,
                      idx_v, buf0, buf1, buf2, gs0, gs1, gs2, ws0, ws1, ws2):
        wid = lax.axis_index("s") * 2 + lax.axis_index("c")
        base = wid * per_w
        w_per_row = ids_hbm.shape[1] // per_w
        pltpu.sync_copy(
            ids_hbm.at[wid // w_per_row,
                       pl.ds((wid % w_per_row) * per_w, per_w)],
            idx_v)

        bufs = (buf0, buf1, buf2)
        gsems = (gs0, gs1, gs2)
        wsems = (ws0, ws1, ws2)
        nbuf = len(bufs)
        gathers = [None] * nbuf
        writes = [None] * nbuf

        def issue_gather(w):
            b = w % nbuf
            gathers[b] = pltpu.async_copy(
                table_hbm.at[idx_v.at[pl.ds(w * GATHER_WINDOW,
                                            GATHER_WINDOW)]],
                bufs[b], gsems[b])

        for w in range(min(nbuf - 1, n_win)):
            issue_gather(w)
        for w in range(n_win):
            b = w % nbuf
            gathers[b].wait()
            wn = w + nbuf - 1
            if wn < n_win:
                bn = wn % nbuf
                if writes[bn] is not None:
                    writes[bn].wait()
                issue_gather(wn)
            writes[b] = pltpu.async_copy(
                bufs[b],
                out_hbm.at[pl.ds(base + w * GATHER_WINDOW, GATHER_WINDOW)],
                wsems[b])
        for wr in writes:
            if wr is not None:
                wr.wait()

    return gather_kernel(embed_table, ids2d)


GATE_BLOCK = 1024


def _tc_gate_kernel(x_ref, w_ref, b_ref, s_ref):
    logits = jnp.dot(x_ref[0], w_ref[...],
                     preferred_element_type=jnp.float32) + b_ref[...]
    s_ref[...] = 2.0 * jax.nn.sigmoid(logits)


def _tc_gate(x, w_t, b_row):
    bsz, seq, _ = x.shape
    t_blocks = seq // GATE_BLOCK
    return pl.pallas_call(
        _tc_gate_kernel,
        out_shape=jax.ShapeDtypeStruct((bsz * seq, KV_HEADS), jnp.float32),
        grid=(bsz, t_blocks),
        in_specs=[
            pl.BlockSpec((1, GATE_BLOCK, GATE_DIM), lambda bi, ti: (bi, ti, 0)),
            pl.BlockSpec((GATE_DIM, KV_HEADS), lambda bi, ti: (0, 0)),
            pl.BlockSpec((1, KV_HEADS), lambda bi, ti: (0, 0)),
        ],
        out_specs=pl.BlockSpec((GATE_BLOCK, KV_HEADS),
                               lambda bi, ti: (bi * t_blocks + ti, 0)),
    )(x, w_t, b_row)


def _tc_scale_kernel(ve_ref, s_ref, out_ref):
    s = s_ref[...]
    for h in range(KV_HEADS):
        out_ref[0, :, h, :] = (ve_ref[:, h * HEAD_DIM:(h + 1) * HEAD_DIM]
                               * s[:, h][:, None])


def _tc_scale_chunk_kernel(prev_ref, ve_ref, s_ref, out_ref):
    del prev_ref
    _tc_scale_kernel(ve_ref, s_ref, out_ref)


N_CHUNKS = 1


def _tc_scale_chunk(out_prev, ve_k, s, k, bsz, seq):
    ck = (bsz * seq) // N_CHUNKS
    t_blocks = ck // TC_BLOCK
    tb_per_b = seq // TC_BLOCK
    base_blk = (k * ck) // TC_BLOCK

    def out_idx(ti):
        flat = base_blk + ti
        return (flat // tb_per_b, flat % tb_per_b, 0, 0)

    ve_spec = pl.BlockSpec((TC_BLOCK, KV), lambda ti: (ti, 0))
    s_spec = pl.BlockSpec((TC_BLOCK, KV_HEADS),
                          lambda ti: (base_blk + ti, 0))
    out_spec = pl.BlockSpec((1, TC_BLOCK, KV_HEADS, HEAD_DIM), out_idx)
    out_shape = jax.ShapeDtypeStruct((bsz, seq, KV_HEADS, HEAD_DIM),
                                     jnp.float32)
    if out_prev is None:
        return pl.pallas_call(
            _tc_scale_kernel,
            out_shape=out_shape,
            grid=(t_blocks,),
            in_specs=[ve_spec, s_spec],
            out_specs=out_spec,
        )(ve_k, s)
    return pl.pallas_call(
        _tc_scale_chunk_kernel,
        out_shape=out_shape,
        grid=(t_blocks,),
        in_specs=[pl.BlockSpec(memory_space=pl.ANY), ve_spec, s_spec],
        out_specs=out_spec,
        input_output_aliases={0: 0},
    )(out_prev, ve_k, s)


def kernel(input_ids, x, layer_idx, embed_table, gate_W, gate_b):
    b, t = input_ids.shape
    n = b * t
    ck = n // N_CHUNKS

    s = _tc_gate(x, gate_W.T, gate_b.reshape(1, KV_HEADS))
    ids_flat = input_ids.reshape(1, n)
    ves = [_sc_gather(embed_table, lax.slice(ids_flat, (0, k * ck),
                                             (1, (k + 1) * ck)))
           for k in range(N_CHUNKS)]
    out = None
    for k in range(N_CHUNKS):
        out = _tc_scale_chunk(out, ves[k], s, k, b, t)
    return out


# speedup vs baseline: 1.0178x; 1.0028x over previous
import functools

import jax
import jax.numpy as jnp
from jax import lax
from jax.experimental import pallas as pl
from jax.experimental.pallas import tpu as pltpu
from jax.experimental.pallas import tpu_sc as plsc

KV_HEADS = 4
HEAD_DIM = 128
GATE_DIM = 128
KV = KV_HEADS * HEAD_DIM

NUM_WORKERS = 32
GATHER_WINDOW = 64
TC_BLOCK = 2048


def _sc_gather(embed_table, ids2d):
    n = ids2d.shape[0] * ids2d.shape[1]
    per_w = n // NUM_WORKERS
    n_win = per_w // GATHER_WINDOW
    mesh = plsc.VectorSubcoreMesh(core_axis_name="c", subcore_axis_name="s")

    @functools.partial(
        pl.kernel,
        out_type=jax.ShapeDtypeStruct((n, KV), embed_table.dtype),
        mesh=mesh,
        scratch_types=[
            pltpu.VMEM((per_w,), jnp.int32),
            pltpu.VMEM((GATHER_WINDOW, KV), jnp.float32),
            pltpu.VMEM((GATHER_WINDOW, KV), jnp.float32),
            pltpu.VMEM((GATHER_WINDOW, KV), jnp.float32),
            pltpu.SemaphoreType.DMA,
            pltpu.SemaphoreType.DMA,
            pltpu.SemaphoreType.DMA,
            pltpu.SemaphoreType.DMA,
            pltpu.SemaphoreType.DMA,
            pltpu.SemaphoreType.DMA,
        ],
    )
    def gather_kernel(table_hbm, ids_hbm, out_hbm,
                      idx_v, buf0, buf1, buf2, gs0, gs1, gs2, ws0, ws1, ws2):
        wid = lax.axis_index("s") * 2 + lax.axis_index("c")
        base = wid * per_w
        w_per_row = ids_hbm.shape[1] // per_w
        pltpu.sync_copy(
            ids_hbm.at[wid // w_per_row,
                       pl.ds((wid % w_per_row) * per_w, per_w)],
            idx_v)

        bufs = (buf0, buf1, buf2)
        gsems = (gs0, gs1, gs2)
        wsems = (ws0, ws1, ws2)
        nbuf = len(bufs)
        gathers = [None] * nbuf
        writes = [None] * nbuf

        def issue_gather(w):
            b = w % nbuf
            gathers[b] = pltpu.async_copy(
                table_hbm.at[idx_v.at[pl.ds(w * GATHER_WINDOW,
                                            GATHER_WINDOW)]],
                bufs[b], gsems[b])

        for w in range(min(nbuf - 1, n_win)):
            issue_gather(w)
        for w in range(n_win):
            b = w % nbuf
            gathers[b].wait()
            wn = w + nbuf - 1
            if wn < n_win:
                bn = wn % nbuf
                if writes[bn] is not None:
                    writes[bn].wait()
                issue_gather(wn)
            writes[b] = pltpu.async_copy(
                bufs[b],
                out_hbm.at[pl.ds(base + w * GATHER_WINDOW, GATHER_WINDOW)],
                wsems[b])
        for wr in writes:
            if wr is not None:
                wr.wait()

    return gather_kernel(embed_table, ids2d)


GATE_BLOCK = 1024


def _tc_gate_kernel(x_ref, w_ref, b_ref, s_ref):
    logits = jnp.dot(x_ref[0], w_ref[...],
                     preferred_element_type=jnp.float32)
    s_ref[...] = 2.0 * jax.nn.sigmoid(logits[:, :KV_HEADS] + b_ref[...])


def _tc_gate(x, w_t, b_row):
    bsz, seq, _ = x.shape
    t_blocks = seq // GATE_BLOCK
    return pl.pallas_call(
        _tc_gate_kernel,
        out_shape=jax.ShapeDtypeStruct((bsz * seq, KV_HEADS), jnp.float32),
        grid=(bsz, t_blocks),
        in_specs=[
            pl.BlockSpec((1, GATE_BLOCK, GATE_DIM), lambda bi, ti: (bi, ti, 0)),
            pl.BlockSpec((GATE_DIM, 128), lambda bi, ti: (0, 0)),
            pl.BlockSpec((1, KV_HEADS), lambda bi, ti: (0, 0)),
        ],
        out_specs=pl.BlockSpec((GATE_BLOCK, KV_HEADS),
                               lambda bi, ti: (bi * t_blocks + ti, 0)),
    )(x, w_t, b_row)


def _tc_scale_kernel(ve_ref, s_ref, out_ref):
    s = s_ref[...]
    for h in range(KV_HEADS):
        out_ref[0, :, h, :] = (ve_ref[:, h * HEAD_DIM:(h + 1) * HEAD_DIM]
                               * s[:, h][:, None])


def _tc_scale_chunk_kernel(prev_ref, ve_ref, s_ref, out_ref):
    del prev_ref
    _tc_scale_kernel(ve_ref, s_ref, out_ref)


N_CHUNKS = 1


def _tc_scale_chunk(out_prev, ve_k, s, k, bsz, seq):
    ck = (bsz * seq) // N_CHUNKS
    t_blocks = ck // TC_BLOCK
    tb_per_b = seq // TC_BLOCK
    base_blk = (k * ck) // TC_BLOCK

    def out_idx(ti):
        flat = base_blk + ti
        return (flat // tb_per_b, flat % tb_per_b, 0, 0)

    ve_spec = pl.BlockSpec((TC_BLOCK, KV), lambda ti: (ti, 0))
    s_spec = pl.BlockSpec((TC_BLOCK, KV_HEADS),
                          lambda ti: (base_blk + ti, 0))
    out_spec = pl.BlockSpec((1, TC_BLOCK, KV_HEADS, HEAD_DIM), out_idx)
    out_shape = jax.ShapeDtypeStruct((bsz, seq, KV_HEADS, HEAD_DIM),
                                     jnp.float32)
    if out_prev is None:
        return pl.pallas_call(
            _tc_scale_kernel,
            out_shape=out_shape,
            grid=(t_blocks,),
            in_specs=[ve_spec, s_spec],
            out_specs=out_spec,
        )(ve_k, s)
    return pl.pallas_call(
        _tc_scale_chunk_kernel,
        out_shape=out_shape,
        grid=(t_blocks,),
        in_specs=[pl.BlockSpec(memory_space=pl.ANY), ve_spec, s_spec],
        out_specs=out_spec,
        input_output_aliases={0: 0},
    )(out_prev, ve_k, s)


def kernel(input_ids, x, layer_idx, embed_table, gate_W, gate_b):
    b, t = input_ids.shape
    n = b * t
    ck = n // N_CHUNKS

    w_pad = jnp.zeros((GATE_DIM, 128), gate_W.dtype).at[:, :KV_HEADS].set(
        gate_W.T)
    s = _tc_gate(x, w_pad, gate_b.reshape(1, KV_HEADS))
    if N_CHUNKS == 1:
        ves = [_sc_gather(embed_table, input_ids)]
    else:
        ids_flat = input_ids.reshape(1, n)
        ves = [_sc_gather(embed_table, lax.slice(ids_flat, (0, k * ck),
                                                 (1, (k + 1) * ck)))
               for k in range(N_CHUNKS)]
    out = None
    for k in range(N_CHUNKS):
        out = _tc_scale_chunk(out, ves[k], s, k, b, t)
    return out
